# baseline (speedup 1.0000x reference)
import functools

import jax
import jax.numpy as jnp
from jax import lax
from jax.experimental import pallas as pl
from jax.experimental.pallas import tpu as pltpu
from jax.experimental.pallas import tpu_sc as plsc

N = 50000
E = 1600000
D_EDGE = 4
HID = 16
C_OUT = 24
W = 16
CNT_COL = 8

NC = 2
NS = 16
NW = NC * NS
EPW = E // NW
CHUNK = 800
TOTCH = E // CHUNK
CPT = -(-TOTCH // NW)
GROUPS = CHUNK // 16

NPAD = 50176
RPT = NPAD // NS
ZROWS = 196
ZCOPIES = RPT // ZROWS

ER = E // 8
KR = 1600

_mesh = plsc.VectorSubcoreMesh(core_axis_name="c", subcore_axis_name="s")
_sc_params = pltpu.CompilerParams(
    needs_layout_passes=False, use_tc_tiling_on_sc=False
)


@functools.partial(
    pl.kernel,
    out_type=(
        jax.ShapeDtypeStruct((NC, NPAD, W), jnp.float32),
        jax.ShapeDtypeStruct((NC, NPAD, W), jnp.float32),
    ),
    mesh=_mesh,
    compiler_params=_sc_params,
    scratch_types=[
        pltpu.VMEM((N,), jnp.float32),
        pltpu.VMEM((CHUNK, W), jnp.float32),
        pltpu.VMEM((CHUNK,), jnp.int32),
        pltpu.VMEM((CHUNK,), jnp.int32),
        pltpu.VMEM((ZROWS, W), jnp.float32),
        pltpu.VMEM_SHARED((NPAD, W), jnp.float32),
    ],
)
def _sc_scatter(x_hbm, lo_hbm, hi_hbm, src_hbm, dst_hbm,
                outlo_hbm, outhi_hbm,
                x_v, msg_v, src_v, dst_v, z_v, acc_sh):
    cid = lax.axis_index("c")
    sid = lax.axis_index("s")
    wid = sid * NC + cid
    base = wid * EPW

    pltpu.sync_copy(x_hbm, x_v)

    zeros16 = jnp.zeros((16,), jnp.float32)

    def zrow(r, c):
        z_v[r, pl.ds(0, 16)] = zeros16
        return c

    lax.fori_loop(0, ZROWS, zrow, 0)

    for phase, (msg_hbm, out_hbm) in enumerate(
        ((lo_hbm, outlo_hbm), (hi_hbm, outhi_hbm))
    ):
        def zcopy(j, c):
            pltpu.sync_copy(
                z_v, acc_sh.at[pl.ds(sid * RPT + j * ZROWS, ZROWS)]
            )
            return c

        lax.fori_loop(0, ZCOPIES, zcopy, 0)
        plsc.subcore_barrier()

        rows16 = lax.iota(jnp.int32, 16)

        def chunk_body(ci, carry):
            gci = wid + NW * ci

            @pl.when(gci < TOTCH)
            def _():
                off = gci * CHUNK
                pltpu.sync_copy(src_hbm.at[pl.ds(off, CHUNK)], src_v)
                pltpu.sync_copy(dst_hbm.at[pl.ds(off, CHUNK)], dst_v)
                iblk = off // BE
                rem = off % BE
                kgrp = rem // KR
                row0 = iblk * KR + rem % KR
                pltpu.sync_copy(
                    msg_hbm.at[pl.ds(row0, CHUNK), pl.ds(kgrp * W, W)],
                    msg_v,
                )

                def grp(gi, c):
                    idx = src_v[pl.ds(gi * 16, 16)]
                    xs = plsc.load_gather(x_v, [idx])
                    for b in range(16):
                        sp = lax.gather(
                            xs,
                            jnp.full((16, 1), b, jnp.int32),
                            lax.GatherDimensionNumbers(
                                offset_dims=(),
                                collapsed_slice_dims=(0,),
                                start_index_map=(0,),
                            ),
                            (1,),
                            mode=lax.GatherScatterMode.PROMISE_IN_BOUNDS,
                        )
                        if phase == 1:
                            sp = jnp.where(rows16 == CNT_COL, 1.0, sp)
                        row = gi * 16 + b
                        msg_v[row, pl.ds(0, 16)] = (
                            msg_v[row, pl.ds(0, 16)] * sp
                        )
                    return c

                lax.fori_loop(0, GROUPS, grp, 0)
                pltpu.sync_copy(msg_v, acc_sh.at[dst_v], add=True)

            return carry

        lax.fori_loop(0, CPT, chunk_body, 0)
        plsc.subcore_barrier()
        pltpu.sync_copy(
            acc_sh.at[pl.ds(sid * RPT, RPT)],
            out_hbm.at[cid, pl.ds(sid * RPT, RPT)],
        )


BE = 12800


def _tc_msg_body(ea_ref, w0_ref, b0_ref, w1p_ref, b1p_ref,
                 lo_ref, hi_ref):
    h = jnp.maximum(
        jnp.dot(ea_ref[...], w0_ref[...], preferred_element_type=jnp.float32)
        + b0_ref[...],
        0.0,
    )
    theta = (
        jnp.dot(h, w1p_ref[...], preferred_element_type=jnp.float32)
        + b1p_ref[...]
    )
    lo_ref[...] = jnp.concatenate(
        [theta[KR * k:KR * (k + 1), :W] for k in range(8)], axis=1
    )
    hi_ref[...] = jnp.concatenate(
        [theta[KR * k:KR * (k + 1), W:] for k in range(8)], axis=1
    )


_tc_msg = pl.pallas_call(
    _tc_msg_body,
    grid=(E // BE,),
    in_specs=[
        pl.BlockSpec((BE, D_EDGE), lambda i: (i, 0)),
        pl.BlockSpec((D_EDGE, HID), lambda i: (0, 0)),
        pl.BlockSpec((1, HID), lambda i: (0, 0)),
        pl.BlockSpec((HID, 2 * W), lambda i: (0, 0)),
        pl.BlockSpec((1, 2 * W), lambda i: (0, 0)),
    ],
    out_specs=(
        pl.BlockSpec((BE // 8, 128), lambda i: (i, 0)),
        pl.BlockSpec((BE // 8, 128), lambda i: (i, 0)),
    ),
    out_shape=(
        jax.ShapeDtypeStruct((ER, 128), jnp.float32),
        jax.ShapeDtypeStruct((ER, 128), jnp.float32),
    ),
)


BN = 2000


def _tc_final_body(plo_ref, phi_ref, x_ref, root_ref, bias_ref, out_ref):
    lo = plo_ref[0] + plo_ref[1]
    hi = phi_ref[0] + phi_ref[1]
    s = jnp.concatenate([lo, hi[:, : C_OUT - W]], axis=1)
    cnt = hi[:, CNT_COL:CNT_COL + 1]
    mean = s / jnp.maximum(cnt, 1.0)
    out_ref[...] = mean + x_ref[...] * root_ref[...] + bias_ref[...]


_tc_final = pl.pallas_call(
    _tc_final_body,
    grid=(N // BN,),
    in_specs=[
        pl.BlockSpec((NC, BN, W), lambda i: (0, i, 0)),
        pl.BlockSpec((NC, BN, W), lambda i: (0, i, 0)),
        pl.BlockSpec((BN, 1), lambda i: (i, 0)),
        pl.BlockSpec((1, C_OUT), lambda i: (0, 0)),
        pl.BlockSpec((1, C_OUT), lambda i: (0, 0)),
    ],
    out_specs=pl.BlockSpec((BN, C_OUT), lambda i: (i, 0)),
    out_shape=jax.ShapeDtypeStruct((N, C_OUT), jnp.float32),
)


def kernel(x, edge_index, edge_attr, w0, b0, w1, b1, root, bias):
    src = edge_index[0]
    dst = edge_index[1]
    w1p = jnp.concatenate(
        [w1, jnp.zeros((HID, 2 * W - C_OUT), jnp.float32)], axis=1
    )
    b1p = jnp.concatenate(
        [
            b1,
            jnp.ones((1,), jnp.float32),
            jnp.zeros((2 * W - C_OUT - 1,), jnp.float32),
        ]
    ).reshape(1, 2 * W)
    theta_lo, theta_hi = _tc_msg(
        edge_attr, w0, b0.reshape(1, HID), w1p, b1p
    )
    p_lo, p_hi = _sc_scatter(x.reshape(N), theta_lo, theta_hi, src, dst)
    out = _tc_final(
        p_lo,
        p_hi,
        x,
        root,
        bias.reshape(1, C_OUT),
    )
    return out


# speedup vs baseline: 1.0235x; 1.0235x over previous
import functools

import jax
import jax.numpy as jnp
from jax import lax
from jax.experimental import pallas as pl
from jax.experimental.pallas import tpu as pltpu
from jax.experimental.pallas import tpu_sc as plsc

N = 50000
E = 1600000
D_EDGE = 4
HID = 16
C_OUT = 24
W = 16
CNT_COL = 8

NC = 2
NS = 16
NW = NC * NS
EPW = E // NW
CHUNK = 800
TOTCH = E // CHUNK
CPT = -(-TOTCH // NW)
GROUPS = CHUNK // 16

NPAD = 50176
RPT = NPAD // NS
ZROWS = 196
ZCOPIES = RPT // ZROWS

ER = E // 8
KR = 1600

_mesh = plsc.VectorSubcoreMesh(core_axis_name="c", subcore_axis_name="s")
_sc_params = pltpu.CompilerParams(
    needs_layout_passes=False, use_tc_tiling_on_sc=False
)


@functools.partial(
    pl.kernel,
    out_type=(
        jax.ShapeDtypeStruct((NC, NPAD, W), jnp.float32),
        jax.ShapeDtypeStruct((NC, NPAD, W), jnp.float32),
    ),
    mesh=_mesh,
    compiler_params=_sc_params,
    scratch_types=[
        pltpu.VMEM((N,), jnp.float32),
        pltpu.VMEM((CHUNK, W), jnp.float32),
        pltpu.VMEM((CHUNK,), jnp.int32),
        pltpu.VMEM((CHUNK,), jnp.int32),
        pltpu.VMEM((ZROWS, W), jnp.float32),
        pltpu.VMEM_SHARED((NPAD, W), jnp.float32),
    ],
)
def _sc_scatter(x_hbm, lo_hbm, hi_hbm, ei_hbm,
                outlo_hbm, outhi_hbm,
                x_v, msg_v, src_v, dst_v, z_v, acc_sh):
    cid = lax.axis_index("c")
    sid = lax.axis_index("s")
    wid = sid * NC + cid
    base = wid * EPW

    pltpu.sync_copy(x_hbm, x_v)

    zeros16 = jnp.zeros((16,), jnp.float32)

    def zrow(r, c):
        z_v[r, pl.ds(0, 16)] = zeros16
        return c

    lax.fori_loop(0, ZROWS, zrow, 0)

    for phase, (msg_hbm, out_hbm) in enumerate(
        ((lo_hbm, outlo_hbm), (hi_hbm, outhi_hbm))
    ):
        def zcopy(j, c):
            pltpu.sync_copy(
                z_v, acc_sh.at[pl.ds(sid * RPT + j * ZROWS, ZROWS)]
            )
            return c

        lax.fori_loop(0, ZCOPIES, zcopy, 0)
        plsc.subcore_barrier()

        rows16 = lax.iota(jnp.int32, 16)

        def chunk_body(ci, carry):
            gci = wid + NW * ci

            @pl.when(gci < TOTCH)
            def _():
                off = gci * CHUNK
                pltpu.sync_copy(ei_hbm.at[0, pl.ds(off, CHUNK)], src_v)
                pltpu.sync_copy(ei_hbm.at[1, pl.ds(off, CHUNK)], dst_v)
                iblk = off // BE
                rem = off % BE
                kgrp = rem // KR
                row0 = iblk * KR + rem % KR
                pltpu.sync_copy(
                    msg_hbm.at[pl.ds(row0, CHUNK), pl.ds(kgrp * W, W)],
                    msg_v,
                )

                def grp(gi, c):
                    idx = src_v[pl.ds(gi * 16, 16)]
                    xs = plsc.load_gather(x_v, [idx])
                    for b in range(16):
                        sp = lax.gather(
                            xs,
                            jnp.full((16, 1), b, jnp.int32),
                            lax.GatherDimensionNumbers(
                                offset_dims=(),
                                collapsed_slice_dims=(0,),
                                start_index_map=(0,),
                            ),
                            (1,),
                            mode=lax.GatherScatterMode.PROMISE_IN_BOUNDS,
                        )
                        if phase == 1:
                            sp = jnp.where(rows16 == CNT_COL, 1.0, sp)
                        row = gi * 16 + b
                        msg_v[row, pl.ds(0, 16)] = (
                            msg_v[row, pl.ds(0, 16)] * sp
                        )
                    return c

                lax.fori_loop(0, GROUPS, grp, 0)
                pltpu.sync_copy(msg_v, acc_sh.at[dst_v], add=True)

            return carry

        lax.fori_loop(0, CPT, chunk_body, 0)
        plsc.subcore_barrier()
        pltpu.sync_copy(
            acc_sh.at[pl.ds(sid * RPT, RPT)],
            out_hbm.at[cid, pl.ds(sid * RPT, RPT)],
        )


BE = 12800


def _tc_msg_body(ea_ref, w0_ref, b0_ref, w1p_ref, b1p_ref,
                 lo_ref, hi_ref):
    h = jnp.maximum(
        jnp.dot(ea_ref[...], w0_ref[...], preferred_element_type=jnp.float32)
        + b0_ref[...],
        0.0,
    )
    theta = (
        jnp.dot(h, w1p_ref[...], preferred_element_type=jnp.float32)
        + b1p_ref[...]
    )
    lo_ref[...] = jnp.concatenate(
        [theta[KR * k:KR * (k + 1), :W] for k in range(8)], axis=1
    )
    hi_ref[...] = jnp.concatenate(
        [theta[KR * k:KR * (k + 1), W:] for k in range(8)], axis=1
    )


_tc_msg = pl.pallas_call(
    _tc_msg_body,
    grid=(E // BE,),
    in_specs=[
        pl.BlockSpec((BE, D_EDGE), lambda i: (i, 0)),
        pl.BlockSpec((D_EDGE, HID), lambda i: (0, 0)),
        pl.BlockSpec((1, HID), lambda i: (0, 0)),
        pl.BlockSpec((HID, 2 * W), lambda i: (0, 0)),
        pl.BlockSpec((1, 2 * W), lambda i: (0, 0)),
    ],
    out_specs=(
        pl.BlockSpec((BE // 8, 128), lambda i: (i, 0)),
        pl.BlockSpec((BE // 8, 128), lambda i: (i, 0)),
    ),
    out_shape=(
        jax.ShapeDtypeStruct((ER, 128), jnp.float32),
        jax.ShapeDtypeStruct((ER, 128), jnp.float32),
    ),
)


BN = 2000


def _tc_final_body(plo_ref, phi_ref, x_ref, root_ref, bias_ref, out_ref):
    lo = plo_ref[0] + plo_ref[1]
    hi = phi_ref[0] + phi_ref[1]
    s = jnp.concatenate([lo, hi[:, : C_OUT - W]], axis=1)
    cnt = hi[:, CNT_COL:CNT_COL + 1]
    mean = s / jnp.maximum(cnt, 1.0)
    out_ref[...] = mean + x_ref[...] * root_ref[...] + bias_ref[...]


_tc_final = pl.pallas_call(
    _tc_final_body,
    grid=(N // BN,),
    in_specs=[
        pl.BlockSpec((NC, BN, W), lambda i: (0, i, 0)),
        pl.BlockSpec((NC, BN, W), lambda i: (0, i, 0)),
        pl.BlockSpec((BN, 1), lambda i: (i, 0)),
        pl.BlockSpec((1, C_OUT), lambda i: (0, 0)),
        pl.BlockSpec((1, C_OUT), lambda i: (0, 0)),
    ],
    out_specs=pl.BlockSpec((BN, C_OUT), lambda i: (i, 0)),
    out_shape=jax.ShapeDtypeStruct((N, C_OUT), jnp.float32),
)


def kernel(x, edge_index, edge_attr, w0, b0, w1, b1, root, bias):
    w1p = jnp.concatenate(
        [w1, jnp.zeros((HID, 2 * W - C_OUT), jnp.float32)], axis=1
    )
    b1p = jnp.concatenate(
        [
            b1,
            jnp.ones((1,), jnp.float32),
            jnp.zeros((2 * W - C_OUT - 1,), jnp.float32),
        ]
    ).reshape(1, 2 * W)
    theta_lo, theta_hi = _tc_msg(
        edge_attr, w0, b0.reshape(1, HID), w1p, b1p
    )
    p_lo, p_hi = _sc_scatter(x.reshape(N), theta_lo, theta_hi, edge_index)
    out = _tc_final(
        p_lo,
        p_hi,
        x,
        root,
        bias.reshape(1, C_OUT),
    )
    return out


# speedup vs baseline: 1.0915x; 1.0664x over previous
import functools

import jax
import jax.numpy as jnp
from jax import lax
from jax.experimental import pallas as pl
from jax.experimental.pallas import tpu as pltpu
from jax.experimental.pallas import tpu_sc as plsc

N = 50000
E = 1600000
D_EDGE = 4
HID = 16
C_OUT = 24
W = 16
CNT_COL = 8

NC = 2
NS = 16
NW = NC * NS
EPW = E // NW
CHUNK = 1600
TOTCH = E // CHUNK
CPT = -(-TOTCH // NW)
GROUPS = CHUNK // 16

NPAD = 50176
RPT = NPAD // NS
ZROWS = 98
ZCOPIES = RPT // ZROWS

ER = E // 8
KR = 1600

_mesh = plsc.VectorSubcoreMesh(core_axis_name="c", subcore_axis_name="s")
_sc_params = pltpu.CompilerParams(
    needs_layout_passes=False, use_tc_tiling_on_sc=False
)


@functools.partial(
    pl.kernel,
    out_type=(
        jax.ShapeDtypeStruct((NC, NPAD, W), jnp.float32),
        jax.ShapeDtypeStruct((NC, NPAD, W), jnp.float32),
    ),
    mesh=_mesh,
    compiler_params=_sc_params,
    scratch_types=[
        pltpu.VMEM((N,), jnp.float32),
        pltpu.VMEM((CHUNK, W), jnp.float32),
        pltpu.VMEM((CHUNK,), jnp.int32),
        pltpu.VMEM((CHUNK,), jnp.int32),
        pltpu.VMEM((ZROWS, W), jnp.float32),
        pltpu.VMEM_SHARED((NPAD, W), jnp.float32),
    ],
)
def _sc_scatter(x_hbm, lo_hbm, hi_hbm, ei_hbm,
                outlo_hbm, outhi_hbm,
                x_v, msg_v, src_v, dst_v, z_v, acc_sh):
    cid = lax.axis_index("c")
    sid = lax.axis_index("s")
    wid = sid * NC + cid
    base = wid * EPW

    pltpu.sync_copy(x_hbm, x_v)

    zeros16 = jnp.zeros((16,), jnp.float32)

    def zrow(r, c):
        z_v[r, pl.ds(0, 16)] = zeros16
        return c

    lax.fori_loop(0, ZROWS, zrow, 0)

    for phase, (msg_hbm, out_hbm) in enumerate(
        ((lo_hbm, outlo_hbm), (hi_hbm, outhi_hbm))
    ):
        def zcopy(j, c):
            pltpu.sync_copy(
                z_v, acc_sh.at[pl.ds(sid * RPT + j * ZROWS, ZROWS)]
            )
            return c

        lax.fori_loop(0, ZCOPIES, zcopy, 0)
        plsc.subcore_barrier()

        rows16 = lax.iota(jnp.int32, 16)

        def chunk_body(ci, carry):
            gci = wid + NW * ci

            @pl.when(gci < TOTCH)
            def _():
                off = gci * CHUNK
                pltpu.sync_copy(ei_hbm.at[0, pl.ds(off, CHUNK)], src_v)
                pltpu.sync_copy(ei_hbm.at[1, pl.ds(off, CHUNK)], dst_v)
                iblk = off // BE
                rem = off % BE
                kgrp = rem // KR
                row0 = iblk * KR + rem % KR
                pltpu.sync_copy(
                    msg_hbm.at[pl.ds(row0, CHUNK), pl.ds(kgrp * W, W)],
                    msg_v,
                )

                def grp(gi, c):
                    idx = src_v[pl.ds(gi * 16, 16)]
                    xs = plsc.load_gather(x_v, [idx])
                    for b in range(16):
                        sp = lax.gather(
                            xs,
                            jnp.full((16, 1), b, jnp.int32),
                            lax.GatherDimensionNumbers(
                                offset_dims=(),
                                collapsed_slice_dims=(0,),
                                start_index_map=(0,),
                            ),
                            (1,),
                            mode=lax.GatherScatterMode.PROMISE_IN_BOUNDS,
                        )
                        if phase == 1:
                            sp = jnp.where(rows16 == CNT_COL, 1.0, sp)
                        row = gi * 16 + b
                        msg_v[row, pl.ds(0, 16)] = (
                            msg_v[row, pl.ds(0, 16)] * sp
                        )
                    return c

                lax.fori_loop(0, GROUPS, grp, 0)
                pltpu.sync_copy(msg_v, acc_sh.at[dst_v], add=True)

            return carry

        lax.fori_loop(0, CPT, chunk_body, 0)
        plsc.subcore_barrier()
        pltpu.sync_copy(
            acc_sh.at[pl.ds(sid * RPT, RPT)],
            out_hbm.at[cid, pl.ds(sid * RPT, RPT)],
        )


BE = 12800


def _tc_msg_body(ea_ref, w0_ref, b0_ref, w1p_ref, b1p_ref,
                 lo_ref, hi_ref):
    h = jnp.maximum(
        jnp.dot(ea_ref[...], w0_ref[...], preferred_element_type=jnp.float32)
        + b0_ref[...],
        0.0,
    )
    theta = (
        jnp.dot(h, w1p_ref[...], preferred_element_type=jnp.float32)
        + b1p_ref[...]
    )
    lo_ref[...] = jnp.concatenate(
        [theta[KR * k:KR * (k + 1), :W] for k in range(8)], axis=1
    )
    hi_ref[...] = jnp.concatenate(
        [theta[KR * k:KR * (k + 1), W:] for k in range(8)], axis=1
    )


_tc_msg = pl.pallas_call(
    _tc_msg_body,
    grid=(E // BE,),
    in_specs=[
        pl.BlockSpec((BE, D_EDGE), lambda i: (i, 0)),
        pl.BlockSpec((D_EDGE, HID), lambda i: (0, 0)),
        pl.BlockSpec((1, HID), lambda i: (0, 0)),
        pl.BlockSpec((HID, 2 * W), lambda i: (0, 0)),
        pl.BlockSpec((1, 2 * W), lambda i: (0, 0)),
    ],
    out_specs=(
        pl.BlockSpec((BE // 8, 128), lambda i: (i, 0)),
        pl.BlockSpec((BE // 8, 128), lambda i: (i, 0)),
    ),
    out_shape=(
        jax.ShapeDtypeStruct((ER, 128), jnp.float32),
        jax.ShapeDtypeStruct((ER, 128), jnp.float32),
    ),
)


BN = 2000


def _tc_final_body(plo_ref, phi_ref, x_ref, root_ref, bias_ref, out_ref):
    lo = plo_ref[0] + plo_ref[1]
    hi = phi_ref[0] + phi_ref[1]
    s = jnp.concatenate([lo, hi[:, : C_OUT - W]], axis=1)
    cnt = hi[:, CNT_COL:CNT_COL + 1]
    mean = s / jnp.maximum(cnt, 1.0)
    out_ref[...] = mean + x_ref[...] * root_ref[...] + bias_ref[...]


_tc_final = pl.pallas_call(
    _tc_final_body,
    grid=(N // BN,),
    in_specs=[
        pl.BlockSpec((NC, BN, W), lambda i: (0, i, 0)),
        pl.BlockSpec((NC, BN, W), lambda i: (0, i, 0)),
        pl.BlockSpec((BN, 1), lambda i: (i, 0)),
        pl.BlockSpec((1, C_OUT), lambda i: (0, 0)),
        pl.BlockSpec((1, C_OUT), lambda i: (0, 0)),
    ],
    out_specs=pl.BlockSpec((BN, C_OUT), lambda i: (i, 0)),
    out_shape=jax.ShapeDtypeStruct((N, C_OUT), jnp.float32),
)


def kernel(x, edge_index, edge_attr, w0, b0, w1, b1, root, bias):
    w1p = jnp.concatenate(
        [w1, jnp.zeros((HID, 2 * W - C_OUT), jnp.float32)], axis=1
    )
    b1p = jnp.concatenate(
        [
            b1,
            jnp.ones((1,), jnp.float32),
            jnp.zeros((2 * W - C_OUT - 1,), jnp.float32),
        ]
    ).reshape(1, 2 * W)
    theta_lo, theta_hi = _tc_msg(
        edge_attr, w0, b0.reshape(1, HID), w1p, b1p
    )
    p_lo, p_hi = _sc_scatter(x.reshape(N), theta_lo, theta_hi, edge_index)
    out = _tc_final(
        p_lo,
        p_hi,
        x,
        root,
        bias.reshape(1, C_OUT),
    )
    return out


# speedup vs baseline: 1.2932x; 1.1849x over previous
import functools

import jax
import jax.numpy as jnp
from jax import lax
from jax.experimental import pallas as pl
from jax.experimental.pallas import tpu as pltpu
from jax.experimental.pallas import tpu_sc as plsc

N = 50000
E = 1600000
D_EDGE = 4
HID = 16
C_OUT = 24
W = 16
CNT_COL = 8

NC = 2
NS = 16
NW = NC * NS
EPW = E // NW
CHUNK = 800
TOTCH = E // CHUNK
CPT = -(-TOTCH // NW)
CPT2 = -(-CPT // 2)
GROUPS = CHUNK // 16

NPAD = 50176
RPT = NPAD // NS
ZROWS = 98
ZCOPIES = RPT // ZROWS

ER = E // 8
KR = 1600

_mesh = plsc.VectorSubcoreMesh(core_axis_name="c", subcore_axis_name="s")
_sc_params = pltpu.CompilerParams(
    needs_layout_passes=False, use_tc_tiling_on_sc=False
)


@functools.partial(
    pl.kernel,
    out_type=(
        jax.ShapeDtypeStruct((NC, NPAD, W), jnp.float32),
        jax.ShapeDtypeStruct((NC, NPAD, W), jnp.float32),
    ),
    mesh=_mesh,
    compiler_params=_sc_params,
    scratch_types=[
        pltpu.VMEM((N,), jnp.float32),
        pltpu.VMEM((CHUNK, W), jnp.float32),
        pltpu.VMEM((CHUNK, W), jnp.float32),
        pltpu.VMEM((CHUNK,), jnp.int32),
        pltpu.VMEM((CHUNK,), jnp.int32),
        pltpu.VMEM((CHUNK,), jnp.int32),
        pltpu.VMEM((CHUNK,), jnp.int32),
        pltpu.VMEM((ZROWS, W), jnp.float32),
        pltpu.VMEM_SHARED((NPAD, W), jnp.float32),
        pltpu.SemaphoreType.DMA,
        pltpu.SemaphoreType.DMA,
    ],
)
def _sc_scatter(x_hbm, lo_hbm, hi_hbm, ei_hbm,
                outlo_hbm, outhi_hbm,
                x_v, msg_v0, msg_v1, src_v0, src_v1, dst_v0, dst_v1,
                z_v, acc_sh, sem0, sem1):
    cid = lax.axis_index("c")
    sid = lax.axis_index("s")
    wid = sid * NC + cid
    base = wid * EPW

    pltpu.sync_copy(x_hbm, x_v)

    zeros16 = jnp.zeros((16,), jnp.float32)

    def zrow(r, c):
        z_v[r, pl.ds(0, 16)] = zeros16
        return c

    lax.fori_loop(0, ZROWS, zrow, 0)

    for phase, (msg_hbm, out_hbm) in enumerate(
        ((lo_hbm, outlo_hbm), (hi_hbm, outhi_hbm))
    ):
        def zcopy(j, c):
            pltpu.sync_copy(
                z_v, acc_sh.at[pl.ds(sid * RPT + j * ZROWS, ZROWS)]
            )
            return c

        lax.fori_loop(0, ZCOPIES, zcopy, 0)
        plsc.subcore_barrier()

        rows16 = lax.iota(jnp.int32, 16)
        slots = (
            (msg_v0, src_v0, dst_v0, sem0),
            (msg_v1, src_v1, dst_v1, sem1),
        )

        def chunk_slices(slot, gci):
            msg_b, src_b, dst_b, sem = slots[slot]
            off = gci * CHUNK
            iblk = off // BE
            rem = off % BE
            kgrp = rem // KR
            row0 = iblk * KR + rem % KR
            return (
                (ei_hbm.at[0, pl.ds(off, CHUNK)], src_b, sem),
                (ei_hbm.at[1, pl.ds(off, CHUNK)], dst_b, sem),
                (
                    msg_hbm.at[pl.ds(row0, CHUNK), pl.ds(kgrp * W, W)],
                    msg_b,
                    sem,
                ),
            )

        def issue_loads(slot, gci):
            for s, d, sem in chunk_slices(slot, gci):
                pltpu.async_copy(s, d, sem)

        def wait_loads(slot, gci):
            for s, d, sem in chunk_slices(slot, gci):
                pltpu.make_async_copy(s, d, sem).wait()

        for s0 in (0, 1):
            gci_p = wid + NW * s0

            @pl.when(gci_p < TOTCH)
            def _(s0=s0, gci_p=gci_p):
                issue_loads(s0, gci_p)

        def chunk_body(ci2, carry):
            for slot in (0, 1):
                gci = wid + NW * (2 * ci2 + slot)

                @pl.when(gci < TOTCH)
                def _(slot=slot, gci=gci):
                    wait_loads(slot, gci)
                    msg_b, src_b, dst_b, sem = slots[slot]

                    def grp(gi, c):
                        idx = src_b[pl.ds(gi * 16, 16)]
                        xs = plsc.load_gather(x_v, [idx])
                        for b in range(16):
                            sp = lax.gather(
                                xs,
                                jnp.full((16, 1), b, jnp.int32),
                                lax.GatherDimensionNumbers(
                                    offset_dims=(),
                                    collapsed_slice_dims=(0,),
                                    start_index_map=(0,),
                                ),
                                (1,),
                                mode=lax.GatherScatterMode.PROMISE_IN_BOUNDS,
                            )
                            if phase == 1:
                                sp = jnp.where(rows16 == CNT_COL, 1.0, sp)
                            row = gi * 16 + b
                            msg_b[row, pl.ds(0, 16)] = (
                                msg_b[row, pl.ds(0, 16)] * sp
                            )
                        return c

                    lax.fori_loop(0, GROUPS, grp, 0)
                    pltpu.sync_copy(msg_b, acc_sh.at[dst_b], add=True)
                    nxt = gci + 2 * NW

                    @pl.when(nxt < TOTCH)
                    def _():
                        issue_loads(slot, nxt)

            return carry

        lax.fori_loop(0, CPT2, chunk_body, 0)
        plsc.subcore_barrier()
        pltpu.sync_copy(
            acc_sh.at[pl.ds(sid * RPT, RPT)],
            out_hbm.at[cid, pl.ds(sid * RPT, RPT)],
        )


BE = 12800


def _tc_msg_body(ea_ref, w0_ref, b0_ref, w1p_ref, b1p_ref,
                 lo_ref, hi_ref):
    h = jnp.maximum(
        jnp.dot(ea_ref[...], w0_ref[...], preferred_element_type=jnp.float32)
        + b0_ref[...],
        0.0,
    )
    theta = (
        jnp.dot(h, w1p_ref[...], preferred_element_type=jnp.float32)
        + b1p_ref[...]
    )
    lo_ref[...] = jnp.concatenate(
        [theta[KR * k:KR * (k + 1), :W] for k in range(8)], axis=1
    )
    hi_ref[...] = jnp.concatenate(
        [theta[KR * k:KR * (k + 1), W:] for k in range(8)], axis=1
    )


_tc_msg = pl.pallas_call(
    _tc_msg_body,
    grid=(E // BE,),
    in_specs=[
        pl.BlockSpec((BE, D_EDGE), lambda i: (i, 0)),
        pl.BlockSpec((D_EDGE, HID), lambda i: (0, 0)),
        pl.BlockSpec((1, HID), lambda i: (0, 0)),
        pl.BlockSpec((HID, 2 * W), lambda i: (0, 0)),
        pl.BlockSpec((1, 2 * W), lambda i: (0, 0)),
    ],
    out_specs=(
        pl.BlockSpec((BE // 8, 128), lambda i: (i, 0)),
        pl.BlockSpec((BE // 8, 128), lambda i: (i, 0)),
    ),
    out_shape=(
        jax.ShapeDtypeStruct((ER, 128), jnp.float32),
        jax.ShapeDtypeStruct((ER, 128), jnp.float32),
    ),
)


BN = 2000


def _tc_final_body(plo_ref, phi_ref, x_ref, root_ref, bias_ref, out_ref):
    lo = plo_ref[0] + plo_ref[1]
    hi = phi_ref[0] + phi_ref[1]
    s = jnp.concatenate([lo, hi[:, : C_OUT - W]], axis=1)
    cnt = hi[:, CNT_COL:CNT_COL + 1]
    mean = s / jnp.maximum(cnt, 1.0)
    out_ref[...] = mean + x_ref[...] * root_ref[...] + bias_ref[...]


_tc_final = pl.pallas_call(
    _tc_final_body,
    grid=(N // BN,),
    in_specs=[
        pl.BlockSpec((NC, BN, W), lambda i: (0, i, 0)),
        pl.BlockSpec((NC, BN, W), lambda i: (0, i, 0)),
        pl.BlockSpec((BN, 1), lambda i: (i, 0)),
        pl.BlockSpec((1, C_OUT), lambda i: (0, 0)),
        pl.BlockSpec((1, C_OUT), lambda i: (0, 0)),
    ],
    out_specs=pl.BlockSpec((BN, C_OUT), lambda i: (i, 0)),
    out_shape=jax.ShapeDtypeStruct((N, C_OUT), jnp.float32),
)


def kernel(x, edge_index, edge_attr, w0, b0, w1, b1, root, bias):
    w1p = jnp.concatenate(
        [w1, jnp.zeros((HID, 2 * W - C_OUT), jnp.float32)], axis=1
    )
    b1p = jnp.concatenate(
        [
            b1,
            jnp.ones((1,), jnp.float32),
            jnp.zeros((2 * W - C_OUT - 1,), jnp.float32),
        ]
    ).reshape(1, 2 * W)
    theta_lo, theta_hi = _tc_msg(
        edge_attr, w0, b0.reshape(1, HID), w1p, b1p
    )
    p_lo, p_hi = _sc_scatter(x.reshape(N), theta_lo, theta_hi, edge_index)
    out = _tc_final(
        p_lo,
        p_hi,
        x,
        root,
        bias.reshape(1, C_OUT),
    )
    return out


# speedup vs baseline: 1.3912x; 1.0758x over previous
import functools

import jax
import jax.numpy as jnp
from jax import lax
from jax.experimental import pallas as pl
from jax.experimental.pallas import tpu as pltpu
from jax.experimental.pallas import tpu_sc as plsc

N = 50000
E = 1600000
D_EDGE = 4
HID = 16
C_OUT = 24
W = 16
CNT_COL = 8

NC = 2
NS = 16
NW = NC * NS
EPW = E // NW
CHUNK = 800
TOTCH = E // CHUNK
CPT = -(-TOTCH // NW)
CPT2 = -(-CPT // 2)
GROUPS = CHUNK // 16

NPAD = 50176
RPT = NPAD // NS
ZROWS = 98
ZCOPIES = RPT // ZROWS

ER = E // 8
KR = 1600

_mesh = plsc.VectorSubcoreMesh(core_axis_name="c", subcore_axis_name="s")
_sc_params = pltpu.CompilerParams(
    needs_layout_passes=False, use_tc_tiling_on_sc=False
)


@functools.partial(
    pl.kernel,
    out_type=(
        jax.ShapeDtypeStruct((NC, NPAD, W), jnp.float32),
        jax.ShapeDtypeStruct((NC, NPAD, W), jnp.float32),
    ),
    mesh=_mesh,
    compiler_params=_sc_params,
    scratch_types=[
        pltpu.VMEM((N,), jnp.float32),
        pltpu.VMEM((CHUNK, W), jnp.float32),
        pltpu.VMEM((CHUNK, W), jnp.float32),
        pltpu.VMEM((CHUNK,), jnp.int32),
        pltpu.VMEM((CHUNK,), jnp.int32),
        pltpu.VMEM((CHUNK,), jnp.int32),
        pltpu.VMEM((CHUNK,), jnp.int32),
        pltpu.VMEM((ZROWS, W), jnp.float32),
        pltpu.VMEM_SHARED((NPAD, W), jnp.float32),
        pltpu.SemaphoreType.DMA,
        pltpu.SemaphoreType.DMA,
    ],
)
def _sc_scatter(x_hbm, lo_hbm, hi_hbm, ei_hbm,
                outlo_hbm, outhi_hbm,
                x_v, msg_v0, msg_v1, src_v0, src_v1, dst_v0, dst_v1,
                z_v, acc_sh, sem0, sem1):
    cid = lax.axis_index("c")
    sid = lax.axis_index("s")
    wid = sid * NC + cid
    base = wid * EPW

    pltpu.sync_copy(x_hbm, x_v)

    zeros16 = jnp.zeros((16,), jnp.float32)

    def zrow(r, c):
        z_v[r, pl.ds(0, 16)] = zeros16
        return c

    lax.fori_loop(0, ZROWS, zrow, 0)

    for phase, (msg_hbm, out_hbm) in enumerate(
        ((lo_hbm, outlo_hbm), (hi_hbm, outhi_hbm))
    ):
        def zcopy(j, c):
            pltpu.sync_copy(
                z_v, acc_sh.at[pl.ds(sid * RPT + j * ZROWS, ZROWS)]
            )
            return c

        lax.fori_loop(0, ZCOPIES, zcopy, 0)
        plsc.subcore_barrier()

        rows16 = lax.iota(jnp.int32, 16)
        slots = (
            (msg_v0, src_v0, dst_v0, sem0),
            (msg_v1, src_v1, dst_v1, sem1),
        )

        def chunk_slices(slot, gci):
            msg_b, src_b, dst_b, sem = slots[slot]
            off = gci * CHUNK
            iblk = off // BE
            rem = off % BE
            kgrp = rem // KR
            row0 = iblk * KR + rem % KR
            return (
                (ei_hbm.at[0, pl.ds(off, CHUNK)], src_b, sem),
                (ei_hbm.at[1, pl.ds(off, CHUNK)], dst_b, sem),
                (
                    msg_hbm.at[pl.ds(row0, CHUNK), pl.ds(kgrp * W, W)],
                    msg_b,
                    sem,
                ),
            )

        def issue_loads(slot, gci):
            for s, d, sem in chunk_slices(slot, gci):
                pltpu.async_copy(s, d, sem)

        def wait_loads(slot, gci):
            for s, d, sem in chunk_slices(slot, gci):
                pltpu.make_async_copy(s, d, sem).wait()

        for s0 in (0, 1):
            gci_p = wid + NW * s0

            @pl.when(gci_p < TOTCH)
            def _(s0=s0, gci_p=gci_p):
                issue_loads(s0, gci_p)

        def chunk_body(ci2, carry):
            for slot in (0, 1):
                gci = wid + NW * (2 * ci2 + slot)

                @pl.when(gci < TOTCH)
                def _(slot=slot, gci=gci):
                    wait_loads(slot, gci)
                    msg_b, src_b, dst_b, sem = slots[slot]

                    def grp(gi, c):
                        idx = src_b[pl.ds(gi * 16, 16)]
                        xs = plsc.load_gather(x_v, [idx])
                        for b in range(16):
                            sp = lax.gather(
                                xs,
                                jnp.full((16, 1), b, jnp.int32),
                                lax.GatherDimensionNumbers(
                                    offset_dims=(),
                                    collapsed_slice_dims=(0,),
                                    start_index_map=(0,),
                                ),
                                (1,),
                                mode=lax.GatherScatterMode.PROMISE_IN_BOUNDS,
                            )
                            if phase == 1:
                                sp = jnp.where(rows16 == CNT_COL, 1.0, sp)
                            row = gi * 16 + b
                            msg_b[row, pl.ds(0, 16)] = (
                                msg_b[row, pl.ds(0, 16)] * sp
                            )
                        return c

                    lax.fori_loop(0, GROUPS, grp, 0)
                    pltpu.sync_copy(msg_b, acc_sh.at[dst_b], add=True)
                    nxt = gci + 2 * NW

                    @pl.when(nxt < TOTCH)
                    def _():
                        issue_loads(slot, nxt)

            return carry

        lax.fori_loop(0, CPT2, chunk_body, 0)
        plsc.subcore_barrier()
        pltpu.sync_copy(
            acc_sh.at[pl.ds(sid * RPT, RPT)],
            out_hbm.at[cid, pl.ds(sid * RPT, RPT)],
        )


BE = 12800


def _tc_msg_body(ea_ref, w0b_ref, b0b_ref, w1lo_ref, b1lo_ref,
                 w1hi_ref, b1hi_ref, lo_ref, hi_ref):
    eab = jnp.concatenate(
        [ea_ref[KR * k:KR * (k + 1), :] for k in range(8)], axis=1
    )
    hbig = jnp.maximum(
        jnp.dot(eab, w0b_ref[...], preferred_element_type=jnp.float32)
        + b0b_ref[...],
        0.0,
    )
    lo_ref[...] = (
        jnp.dot(hbig, w1lo_ref[...], preferred_element_type=jnp.float32)
        + b1lo_ref[...]
    )
    hi_ref[...] = (
        jnp.dot(hbig, w1hi_ref[...], preferred_element_type=jnp.float32)
        + b1hi_ref[...]
    )


_tc_msg = pl.pallas_call(
    _tc_msg_body,
    grid=(E // BE,),
    in_specs=[
        pl.BlockSpec((BE, D_EDGE), lambda i: (i, 0)),
        pl.BlockSpec((8 * D_EDGE, 128), lambda i: (0, 0)),
        pl.BlockSpec((1, 128), lambda i: (0, 0)),
        pl.BlockSpec((128, 128), lambda i: (0, 0)),
        pl.BlockSpec((1, 128), lambda i: (0, 0)),
        pl.BlockSpec((128, 128), lambda i: (0, 0)),
        pl.BlockSpec((1, 128), lambda i: (0, 0)),
    ],
    out_specs=(
        pl.BlockSpec((BE // 8, 128), lambda i: (i, 0)),
        pl.BlockSpec((BE // 8, 128), lambda i: (i, 0)),
    ),
    out_shape=(
        jax.ShapeDtypeStruct((ER, 128), jnp.float32),
        jax.ShapeDtypeStruct((ER, 128), jnp.float32),
    ),
)


BN = 2000


def _tc_final_body(plo_ref, phi_ref, x_ref, root_ref, bias_ref, out_ref):
    lo = plo_ref[0] + plo_ref[1]
    hi = phi_ref[0] + phi_ref[1]
    s = jnp.concatenate([lo, hi[:, : C_OUT - W]], axis=1)
    cnt = hi[:, CNT_COL:CNT_COL + 1]
    mean = s / jnp.maximum(cnt, 1.0)
    out_ref[...] = mean + x_ref[...] * root_ref[...] + bias_ref[...]


_tc_final = pl.pallas_call(
    _tc_final_body,
    grid=(N // BN,),
    in_specs=[
        pl.BlockSpec((NC, BN, W), lambda i: (0, i, 0)),
        pl.BlockSpec((NC, BN, W), lambda i: (0, i, 0)),
        pl.BlockSpec((BN, 1), lambda i: (i, 0)),
        pl.BlockSpec((1, C_OUT), lambda i: (0, 0)),
        pl.BlockSpec((1, C_OUT), lambda i: (0, 0)),
    ],
    out_specs=pl.BlockSpec((BN, C_OUT), lambda i: (i, 0)),
    out_shape=jax.ShapeDtypeStruct((N, C_OUT), jnp.float32),
)


def kernel(x, edge_index, edge_attr, w0, b0, w1, b1, root, bias):
    eye8 = jnp.eye(8, dtype=jnp.float32)
    w0big = jnp.kron(eye8, w0)
    b0big = jnp.tile(b0, 8).reshape(1, 128)
    w1lo = jnp.kron(eye8, w1[:, :W])
    w1hi = jnp.kron(
        eye8,
        jnp.concatenate(
            [w1[:, W:], jnp.zeros((HID, 2 * W - C_OUT), jnp.float32)],
            axis=1,
        ),
    )
    b1lo = jnp.tile(b1[:W], 8).reshape(1, 128)
    b1hi = jnp.tile(
        jnp.concatenate(
            [
                b1[W:],
                jnp.ones((1,), jnp.float32),
                jnp.zeros((2 * W - C_OUT - 1,), jnp.float32),
            ]
        ),
        8,
    ).reshape(1, 128)
    theta_lo, theta_hi = _tc_msg(
        edge_attr, w0big, b0big, w1lo, b1lo, w1hi, b1hi
    )
    p_lo, p_hi = _sc_scatter(x.reshape(N), theta_lo, theta_hi, edge_index)
    out = _tc_final(
        p_lo,
        p_hi,
        x,
        root,
        bias.reshape(1, C_OUT),
    )
    return out
